# baseline (speedup 1.0000x reference)
import jax
import jax.numpy as jnp
from jax import lax
from jax.experimental import pallas as pl
from jax.experimental.pallas import tpu as pltpu
from jax.experimental.pallas import tpu_sc as plsc

_NC = 2
_NS = 16
_NW = _NC * _NS
_L = 16

_CHUNK = 6144
_NBUF = 4


def _make_tc_body(n, d):
    g_full = n // _CHUNK
    rem = n - g_full * _CHUNK

    def body(const_ref, at_hbm, feat_hbm, out_hbm, nf_hbm,
             atv, nfv, fbuf, obuf, ftail, otail,
             at_sem, nf_sem, in_sems, out_sems, tin_sem, tout_sem):
        def in_copy(i, slot):
            return pltpu.make_async_copy(
                feat_hbm.at[pl.ds(i * _CHUNK, _CHUNK), :], fbuf.at[slot],
                in_sems.at[slot])

        def out_copy(i, slot):
            return pltpu.make_async_copy(
                obuf.at[slot], out_hbm.at[pl.ds(i * _CHUNK, _CHUNK), :],
                out_sems.at[slot])

        in_copy(0, 0).start()
        at_dma = pltpu.make_async_copy(at_hbm, atv, at_sem)
        at_dma.start()
        if rem:
            tail_in = pltpu.make_async_copy(
                feat_hbm.at[pl.ds(g_full * _CHUNK, rem), :], ftail, tin_sem)
            tail_in.start()
        for s in range(1, min(_NBUF, g_full)):
            in_copy(s, s).start()
        at_dma.wait()

        c0 = const_ref[0]
        c1 = const_ref[1]
        c2 = const_ref[2]
        c3 = const_ref[3]

        def factors(row0):
            at = atv[pl.ds(row0, _CHUNK)]
            nf1 = jnp.where(
                at == 0, c0,
                jnp.where(at == 1, c1, jnp.where(at == 2, c2, c3)))
            nfv[pl.ds(row0, _CHUNK)] = nf1
            return jnp.reshape(nf1, (_CHUNK, 1))

        def step(i, carry):
            slot = lax.rem(i, _NBUF)
            in_copy(i, slot).wait()

            @pl.when(i >= _NBUF)
            def _():
                out_copy(lax.max(i - _NBUF, 0), slot).wait()

            obuf[slot] = fbuf[slot] * factors(i * _CHUNK)
            out_copy(i, slot).start()

            @pl.when(i + _NBUF < g_full)
            def _():
                in_copy(i + _NBUF, slot).start()

            return carry

        lax.fori_loop(0, g_full, step, 0, unroll=False)

        if rem:
            tail_in.wait()
            nf = factors(g_full * _CHUNK)
            otail[...] = ftail[...] * nf[:rem]
            pltpu.make_async_copy(
                otail, out_hbm.at[pl.ds(g_full * _CHUNK, rem), :],
                tout_sem).start()

        for s in range(min(_NBUF, g_full)):
            i = g_full - min(_NBUF, g_full) + s
            out_copy(i, i % _NBUF).wait()

        nf_dma = pltpu.make_async_copy(nfv, nf_hbm, nf_sem)
        nf_dma.start()
        nf_dma.wait()

        if rem:
            pltpu.make_async_copy(
                otail, out_hbm.at[pl.ds(g_full * _CHUNK, rem), :],
                tout_sem).wait()

    return body, g_full, rem


def _tc_scale(node_features, at_padded_1d, const_flat):
    n, d = node_features.shape
    body, g_full, rem = _make_tc_body(n, d)
    tail_rows = max(rem, 8)
    return pl.pallas_call(
        body,
        in_specs=[
            pl.BlockSpec(memory_space=pltpu.SMEM),
            pl.BlockSpec(memory_space=pl.ANY),
            pl.BlockSpec(memory_space=pl.ANY),
        ],
        out_specs=[pl.BlockSpec(memory_space=pl.ANY),
                   pl.BlockSpec(memory_space=pl.ANY)],
        out_shape=[jax.ShapeDtypeStruct((n, d), node_features.dtype),
                   jax.ShapeDtypeStruct((at_padded_1d.shape[0],),
                                        jnp.float32)],
        scratch_shapes=[
            pltpu.VMEM((at_padded_1d.shape[0],), jnp.int32),
            pltpu.VMEM((at_padded_1d.shape[0],), jnp.float32),
            pltpu.VMEM((_NBUF, _CHUNK, d), node_features.dtype),
            pltpu.VMEM((_NBUF, _CHUNK, d), node_features.dtype),
            pltpu.VMEM((tail_rows, d), node_features.dtype),
            pltpu.VMEM((tail_rows, d), node_features.dtype),
            pltpu.SemaphoreType.DMA,
            pltpu.SemaphoreType.DMA,
            pltpu.SemaphoreType.DMA((_NBUF,)),
            pltpu.SemaphoreType.DMA((_NBUF,)),
            pltpu.SemaphoreType.DMA,
            pltpu.SemaphoreType.DMA,
        ],
    )(const_flat, at_padded_1d, node_features)


def _sc_norm_factor(at_padded, table16, bpw):
    npad = at_padded.shape[0]
    mesh = plsc.VectorSubcoreMesh(core_axis_name="c", subcore_axis_name="s")

    def body(at_hbm, tbl_hbm, out_hbm, idx_v, tbl_v, out_v):
        wid = lax.axis_index("s") * _NC + lax.axis_index("c")
        base = wid * bpw
        pltpu.sync_copy(at_hbm.at[pl.ds(base, bpw)], idx_v)
        pltpu.sync_copy(tbl_hbm, tbl_v)
        tbl = tbl_v[...]
        c0 = tbl[0]
        c1 = tbl[1]
        c2 = tbl[2]
        c3 = tbl[3]

        def step(i, carry):
            idx = idx_v[pl.ds(i * _L, _L)]
            out_v[pl.ds(i * _L, _L)] = jnp.where(
                idx == 0,
                c0,
                jnp.where(idx == 1, c1, jnp.where(idx == 2, c2, c3)),
            )
            return carry

        lax.fori_loop(0, bpw // _L, step, 0, unroll=8)
        pltpu.sync_copy(out_v, out_hbm.at[pl.ds(base, bpw)])

    return pl.kernel(
        body,
        out_type=jax.ShapeDtypeStruct((npad,), jnp.float32),
        mesh=mesh,
        scratch_types=[
            pltpu.VMEM((bpw,), jnp.int32),
            pltpu.VMEM((_L,), jnp.float32),
            pltpu.VMEM((bpw,), jnp.float32),
        ],
    )(at_padded, table16)


def kernel(node_features, atom_types, norm_const):
    n, _ = node_features.shape
    at32 = atom_types.astype(jnp.int32)
    const_flat = norm_const.reshape(-1)

    chunk = _NW * _L
    npad = ((n + _CHUNK - 1) // _CHUNK) * _CHUNK
    npad = ((npad + chunk - 1) // chunk) * chunk
    at_padded = jnp.pad(at32, (0, npad - n))

    out_features, nf_padded = _tc_scale(node_features, at_padded, const_flat)
    norm_factor = nf_padded[:n].reshape(n, 1)

    return out_features, norm_factor


# speedup vs baseline: 1.0024x; 1.0024x over previous
import jax
import jax.numpy as jnp
from jax import lax
from jax.experimental import pallas as pl
from jax.experimental.pallas import tpu as pltpu

_CHUNK = 6144
_NBUF = 4


def _make_tc_body(n, d):
    g_full = n // _CHUNK
    rem = n - g_full * _CHUNK

    def body(const_ref, at_hbm, feat_hbm, out_hbm, nf_hbm,
             atv, nfv, fbuf, obuf, ftail, otail,
             at_sem, nf_sem, in_sems, out_sems, tin_sem, tout_sem):
        def in_copy(i, slot):
            return pltpu.make_async_copy(
                feat_hbm.at[pl.ds(i * _CHUNK, _CHUNK), :], fbuf.at[slot],
                in_sems.at[slot])

        def out_copy(i, slot):
            return pltpu.make_async_copy(
                obuf.at[slot], out_hbm.at[pl.ds(i * _CHUNK, _CHUNK), :],
                out_sems.at[slot])

        in_copy(0, 0).start()
        at_dma = pltpu.make_async_copy(at_hbm, atv, at_sem)
        at_dma.start()
        if rem:
            tail_in = pltpu.make_async_copy(
                feat_hbm.at[pl.ds(g_full * _CHUNK, rem), :], ftail, tin_sem)
            tail_in.start()
        for s in range(1, min(_NBUF, g_full)):
            in_copy(s, s).start()
        at_dma.wait()

        c0 = const_ref[0]
        c1 = const_ref[1]
        c2 = const_ref[2]
        c3 = const_ref[3]

        def factors(row0):
            at = atv[pl.ds(row0, _CHUNK)]
            nf1 = jnp.where(
                at == 0, c0,
                jnp.where(at == 1, c1, jnp.where(at == 2, c2, c3)))
            nfv[pl.ds(row0, _CHUNK)] = nf1
            return jnp.reshape(nf1, (_CHUNK, 1))

        def step(i, carry):
            slot = lax.rem(i, _NBUF)
            in_copy(i, slot).wait()

            @pl.when(i >= _NBUF)
            def _():
                out_copy(lax.max(i - _NBUF, 0), slot).wait()

            obuf[slot] = fbuf[slot] * factors(i * _CHUNK)
            out_copy(i, slot).start()

            @pl.when(i + _NBUF < g_full)
            def _():
                in_copy(i + _NBUF, slot).start()

            return carry

        lax.fori_loop(0, g_full, step, 0, unroll=False)

        if rem:
            tail_in.wait()
            nf = factors(g_full * _CHUNK)
            otail[...] = ftail[...] * nf[:rem]
            pltpu.make_async_copy(
                otail, out_hbm.at[pl.ds(g_full * _CHUNK, rem), :],
                tout_sem).start()

        for s in range(min(_NBUF, g_full)):
            i = g_full - min(_NBUF, g_full) + s
            out_copy(i, i % _NBUF).wait()

        nf_dma = pltpu.make_async_copy(nfv, nf_hbm, nf_sem)
        nf_dma.start()
        nf_dma.wait()

        if rem:
            pltpu.make_async_copy(
                otail, out_hbm.at[pl.ds(g_full * _CHUNK, rem), :],
                tout_sem).wait()

    return body, g_full, rem


def _tc_scale(node_features, at_padded_1d, const_flat):
    n, d = node_features.shape
    body, g_full, rem = _make_tc_body(n, d)
    tail_rows = max(rem, 8)
    return pl.pallas_call(
        body,
        in_specs=[
            pl.BlockSpec(memory_space=pltpu.SMEM),
            pl.BlockSpec(memory_space=pl.ANY),
            pl.BlockSpec(memory_space=pl.ANY),
        ],
        out_specs=[pl.BlockSpec(memory_space=pl.ANY),
                   pl.BlockSpec(memory_space=pl.ANY)],
        out_shape=[jax.ShapeDtypeStruct((n, d), node_features.dtype),
                   jax.ShapeDtypeStruct((at_padded_1d.shape[0],),
                                        jnp.float32)],
        scratch_shapes=[
            pltpu.VMEM((at_padded_1d.shape[0],), jnp.int32),
            pltpu.VMEM((at_padded_1d.shape[0],), jnp.float32),
            pltpu.VMEM((_NBUF, _CHUNK, d), node_features.dtype),
            pltpu.VMEM((_NBUF, _CHUNK, d), node_features.dtype),
            pltpu.VMEM((tail_rows, d), node_features.dtype),
            pltpu.VMEM((tail_rows, d), node_features.dtype),
            pltpu.SemaphoreType.DMA,
            pltpu.SemaphoreType.DMA,
            pltpu.SemaphoreType.DMA((_NBUF,)),
            pltpu.SemaphoreType.DMA((_NBUF,)),
            pltpu.SemaphoreType.DMA,
            pltpu.SemaphoreType.DMA,
        ],
    )(const_flat, at_padded_1d, node_features)


def kernel(node_features, atom_types, norm_const):
    n, _ = node_features.shape
    at32 = atom_types.astype(jnp.int32)
    const_flat = norm_const.reshape(-1)

    npad = ((n + _CHUNK - 1) // _CHUNK) * _CHUNK
    at_padded = jnp.pad(at32, (0, npad - n))

    out_features, nf_padded = _tc_scale(node_features, at_padded, const_flat)
    norm_factor = nf_padded[:n].reshape(n, 1)

    return out_features, norm_factor
